# baseline (speedup 1.0000x reference)
import jax
import jax.numpy as jnp
from jax.experimental import pallas as pl
from jax.experimental.pallas import tpu as pltpu

_BM = 6144


def _rowmm_kernel(a_ref, x_ref, o_ref):
    xb = x_ref[...].astype(jnp.bfloat16)
    ab = a_ref[...].astype(jnp.bfloat16)
    o_ref[...] = jax.lax.dot_general(
        xb, ab,
        dimension_numbers=(((1,), (1,)), ((), ())),
        preferred_element_type=jnp.float32)


def kernel(x, A):
    N, C, W, L = x.shape
    V = A.shape[0]
    M = N * C * L

    xt = jnp.transpose(x, (0, 1, 3, 2)).reshape(M, W)

    bm = _BM
    flops = 2 * M * W * V
    bytes_accessed = (M * W + M * V) * 4 + W * V * 2

    out2 = pl.pallas_call(
        _rowmm_kernel,
        out_shape=jax.ShapeDtypeStruct((M, V), jnp.float32),
        grid=(M // bm,),
        in_specs=[
            pl.BlockSpec((V, W), lambda i: (0, 0)),
            pl.BlockSpec((bm, W), lambda i: (i, 0)),
        ],
        out_specs=pl.BlockSpec((bm, V), lambda i: (i, 0)),
        compiler_params=pltpu.CompilerParams(
            dimension_semantics=("parallel",),
            vmem_limit_bytes=100 * 1024 * 1024,
        ),
        cost_estimate=pl.CostEstimate(
            flops=flops, transcendentals=0, bytes_accessed=bytes_accessed),
    )(A, xt)
    return jnp.transpose(out2.reshape(N, C, L, V), (0, 1, 3, 2))


# speedup vs baseline: 1.0538x; 1.0538x over previous
import jax
import jax.numpy as jnp
from jax.experimental import pallas as pl
from jax.experimental.pallas import tpu as pltpu

_BM = 12288


def _rowmm_kernel(a_ref, x_ref, o_ref):
    xb = x_ref[...].astype(jnp.bfloat16)
    ab = a_ref[...].astype(jnp.bfloat16)
    o_ref[...] = jax.lax.dot_general(
        xb, ab,
        dimension_numbers=(((1,), (1,)), ((), ())),
        preferred_element_type=jnp.float32)


def kernel(x, A):
    N, C, W, L = x.shape
    V = A.shape[0]
    M = N * C * L

    xt = jnp.transpose(x, (0, 1, 3, 2)).reshape(M, W)

    bm = _BM
    flops = 2 * M * W * V
    bytes_accessed = (M * W + M * V) * 4 + W * V * 2

    out2 = pl.pallas_call(
        _rowmm_kernel,
        out_shape=jax.ShapeDtypeStruct((M, V), jnp.float32),
        grid=(M // bm,),
        in_specs=[
            pl.BlockSpec((V, W), lambda i: (0, 0)),
            pl.BlockSpec((bm, W), lambda i: (i, 0)),
        ],
        out_specs=pl.BlockSpec((bm, V), lambda i: (i, 0)),
        compiler_params=pltpu.CompilerParams(
            dimension_semantics=("parallel",),
            vmem_limit_bytes=100 * 1024 * 1024,
        ),
        cost_estimate=pl.CostEstimate(
            flops=flops, transcendentals=0, bytes_accessed=bytes_accessed),
    )(A, xt)
    return jnp.transpose(out2.reshape(N, C, L, V), (0, 1, 3, 2))


# speedup vs baseline: 1.1891x; 1.1283x over previous
import jax
import jax.numpy as jnp
from jax.experimental import pallas as pl
from jax.experimental.pallas import tpu as pltpu

_BM = 24576


def _rowmm_kernel(a_ref, x_ref, o_ref):
    xb = x_ref[...].astype(jnp.bfloat16)
    ab = a_ref[...].astype(jnp.bfloat16)
    o_ref[...] = jax.lax.dot_general(
        xb, ab,
        dimension_numbers=(((1,), (1,)), ((), ())),
        preferred_element_type=jnp.float32)


def kernel(x, A):
    N, C, W, L = x.shape
    V = A.shape[0]
    M = N * C * L

    xt = jnp.transpose(x, (0, 1, 3, 2)).reshape(M, W)

    bm = _BM
    flops = 2 * M * W * V
    bytes_accessed = (M * W + M * V) * 4 + W * V * 2

    out2 = pl.pallas_call(
        _rowmm_kernel,
        out_shape=jax.ShapeDtypeStruct((M, V), jnp.float32),
        grid=(M // bm,),
        in_specs=[
            pl.BlockSpec((V, W), lambda i: (0, 0)),
            pl.BlockSpec((bm, W), lambda i: (i, 0)),
        ],
        out_specs=pl.BlockSpec((bm, V), lambda i: (i, 0)),
        compiler_params=pltpu.CompilerParams(
            dimension_semantics=("parallel",),
            vmem_limit_bytes=100 * 1024 * 1024,
        ),
        cost_estimate=pl.CostEstimate(
            flops=flops, transcendentals=0, bytes_accessed=bytes_accessed),
    )(A, xt)
    return jnp.transpose(out2.reshape(N, C, L, V), (0, 1, 3, 2))


# speedup vs baseline: 1.1929x; 1.0032x over previous
import jax
import jax.numpy as jnp
from jax.experimental import pallas as pl
from jax.experimental.pallas import tpu as pltpu


def _rowmm_kernel(a_ref, x_ref, o_ref):
    xb = x_ref[...].astype(jnp.bfloat16)
    ab = a_ref[...].astype(jnp.bfloat16)
    o_ref[...] = jax.lax.dot_general(
        xb, ab,
        dimension_numbers=(((1,), (1,)), ((), ())),
        preferred_element_type=jnp.float32)


def kernel(x, A):
    N, C, W, L = x.shape
    V = A.shape[0]
    M = N * C * L

    xt = jnp.transpose(x, (0, 1, 3, 2)).reshape(M, W)

    bm = M // 2
    flops = 2 * M * W * V
    bytes_accessed = (M * W + M * V) * 4 + W * V * 2

    out2 = pl.pallas_call(
        _rowmm_kernel,
        out_shape=jax.ShapeDtypeStruct((M, V), jnp.float32),
        grid=(M // bm,),
        in_specs=[
            pl.BlockSpec((V, W), lambda i: (0, 0)),
            pl.BlockSpec((bm, W), lambda i: (i, 0)),
        ],
        out_specs=pl.BlockSpec((bm, V), lambda i: (i, 0)),
        compiler_params=pltpu.CompilerParams(
            dimension_semantics=("parallel",),
            vmem_limit_bytes=100 * 1024 * 1024,
        ),
        cost_estimate=pl.CostEstimate(
            flops=flops, transcendentals=0, bytes_accessed=bytes_accessed),
    )(A, xt)
    return jnp.transpose(out2.reshape(N, C, L, V), (0, 1, 3, 2))
